# baseline (speedup 1.0000x reference)
import functools

import jax
import jax.numpy as jnp
import numpy as np
from jax import lax
from jax.experimental import pallas as pl
from jax.experimental.pallas import tpu as pltpu
from jax.experimental.pallas import tpu_sc as plsc

STRIDES = [8, 16, 32, 64, 128]
SIZES = [(128, 128), (64, 64), (32, 32), (16, 16), (8, 8)]
B = 4
NC = 80
TCOLS = 96
NANCH = sum(h * w for h, w in SIZES)
NPAD = 22528
NSEL = 1024
NMS_PRE = 1000
SCORE_THR = 0.05
IOU_THR = 0.5
MAX_PER_IMG = 100
MAXRATIO_G = 13.815511
MAXRATIO_P = float(abs(np.log(16.0 / 1000.0)))
LOC_THR = 0.01


def _k1_body(base, img_h, img_w, cls_ref, bbox_ref, shape_ref, loc_ref,
             px_ref, py_ref, maxsc_ref, tab_ref):
    cls = cls_ref[0]
    maxlogit = jnp.max(cls, axis=0)[None, :]
    loc_s = jax.nn.sigmoid(loc_ref[0])
    mask = loc_s >= LOC_THR
    ss = jax.nn.sigmoid(jax.nn.sigmoid(maxlogit))
    maxsc_ref[0] = jnp.where(mask, ss, 0.0)

    px = px_ref[...]
    py = py_ref[...]
    dw = jnp.clip(shape_ref[0, 0:1, :] * 0.14, -MAXRATIO_G, MAXRATIO_G)
    dh = jnp.clip(shape_ref[0, 1:2, :] * 0.14, -MAXRATIO_G, MAXRATIO_G)
    gw = base * jnp.exp(dw)
    gh = base * jnp.exp(dh)
    d2x = bbox_ref[0, 0:1, :]
    d2y = bbox_ref[0, 1:2, :]
    d2w = jnp.clip(bbox_ref[0, 2:3, :], -MAXRATIO_P, MAXRATIO_P)
    d2h = jnp.clip(bbox_ref[0, 3:4, :], -MAXRATIO_P, MAXRATIO_P)
    g2w = gw * jnp.exp(d2w)
    g2h = gh * jnp.exp(d2h)
    g2x = px + gw * d2x
    g2y = py + gh * d2y
    x1 = jnp.clip(g2x - 0.5 * g2w, 0.0, img_w)
    y1 = jnp.clip(g2y - 0.5 * g2h, 0.0, img_h)
    x2 = jnp.clip(g2x + 0.5 * g2w, 0.0, img_w)
    y2 = jnp.clip(g2y + 0.5 * g2h, 0.0, img_h)
    boxes = jnp.concatenate([x1, y1, x2, y2], axis=0)
    ch = cls.shape[1]
    row = jnp.concatenate(
        [jnp.transpose(boxes, (1, 0)),
         jnp.transpose(cls, (1, 0)),
         jnp.zeros((ch, TCOLS - 4 - NC), jnp.float32)], axis=1)
    tab_ref[0] = row


def _k1_level(lvl, cls, bbox, shp, loc, img_h, img_w):
    H, W = SIZES[lvl]
    hw = H * W
    ch = min(hw, 512)
    grid = (B, hw // ch)
    stride = STRIDES[lvl]
    xs = (np.arange(hw) % W).astype(np.float32) * stride
    ys = (np.arange(hw) // W).astype(np.float32) * stride
    px = jnp.asarray(xs)[None, :]
    py = jnp.asarray(ys)[None, :]
    base = float(stride * 4.0)
    out = pl.pallas_call(
        functools.partial(_k1_body, base, float(img_h), float(img_w)),
        grid=grid,
        in_specs=[
            pl.BlockSpec((1, NC, ch), lambda b, i: (b, 0, i)),
            pl.BlockSpec((1, 4, ch), lambda b, i: (b, 0, i)),
            pl.BlockSpec((1, 2, ch), lambda b, i: (b, 0, i)),
            pl.BlockSpec((1, 1, ch), lambda b, i: (b, 0, i)),
            pl.BlockSpec((1, ch), lambda b, i: (0, i)),
            pl.BlockSpec((1, ch), lambda b, i: (0, i)),
        ],
        out_specs=[
            pl.BlockSpec((1, 1, ch), lambda b, i: (b, 0, i)),
            pl.BlockSpec((1, ch, TCOLS), lambda b, i: (b, i, 0)),
        ],
        out_shape=[
            jax.ShapeDtypeStruct((B, 1, hw), jnp.float32),
            jax.ShapeDtypeStruct((B, hw, TCOLS), jnp.float32),
        ],
    )(cls.reshape(B, NC, hw), bbox.reshape(B, 4, hw),
      shp.reshape(B, 2, hw), loc.reshape(B, 1, hw), px, py)
    return out[0].reshape(B, hw), out[1]


def _bisect_meta(vals, gidx, k, nchunks):
    bits = jax.lax.bitcast_convert_type(vals, jnp.int32)
    R = vals.shape[0]

    def vstep(_, c):
        lo, hi = c
        mid = lo + (hi - lo + 1) // 2
        cnt = jnp.sum((bits >= mid).astype(jnp.int32))
        p = cnt >= k
        return jnp.where(p, mid, lo), jnp.where(p, hi, mid - 1)

    lo0 = jnp.int32(-1082130432)
    hi0 = jnp.int32(0x3F400000)
    t, _ = jax.lax.fori_loop(0, 31, vstep, (lo0, hi0))

    n_gt = jnp.sum((bits > t).astype(jnp.int32))
    need = k - n_gt
    eq = bits == t

    def istep(_, c):
        lo, hi = c
        mid = (lo + hi) // 2
        cnt = jnp.sum((eq & (gidx <= mid)).astype(jnp.int32))
        p = cnt >= need
        return jnp.where(p, lo, mid), jnp.where(p, mid, hi)

    lo0i = jnp.int32(-1)
    hi0i = jnp.int32(R * vals.shape[1] - 1)
    _, cut = jax.lax.fori_loop(0, 18, istep, (lo0i, hi0i))

    sel = (bits > t) | (eq & (gidx <= cut))
    cr = R // nchunks
    rows = [jnp.full((1, 16), t, jnp.int32), jnp.full((1, 16), cut, jnp.int32)]
    basev = jnp.int32(0)
    for j in range(nchunks):
        rows.append(jnp.full((1, 16), basev, jnp.int32))
        basev = basev + jnp.sum(sel[j * cr:(j + 1) * cr].astype(jnp.int32))
    return jnp.concatenate(rows, axis=0)


def _k2_body(k, nchunks, vals_ref, meta_ref):
    v = vals_ref[0]
    R = v.shape[0]
    gidx = (jax.lax.broadcasted_iota(jnp.int32, (R, 128), 0) * 128
            + jax.lax.broadcasted_iota(jnp.int32, (R, 128), 1))
    meta_ref[0] = _bisect_meta(v, gidx, k, nchunks)


def _k2_call(vals):
    R = vals.shape[1]
    return pl.pallas_call(
        functools.partial(_k2_body, NSEL, 8),
        grid=(B,),
        in_specs=[pl.BlockSpec((1, R, 128), lambda b: (b, 0, 0))],
        out_specs=pl.BlockSpec((1, 10, 16), lambda b: (b, 0, 0)),
        out_shape=jax.ShapeDtypeStruct((B, 10, 16), jnp.int32),
    )(vals)


def _sort_net(key, idx, extras):
    vals = [key, idx] + list(extras)
    cur = 'A'

    def posof(layout, R, C):
        if layout == 'A':
            return (jax.lax.broadcasted_iota(jnp.int32, (R, C), 0) * 128
                    + jax.lax.broadcasted_iota(jnp.int32, (R, C), 1))
        return (jax.lax.broadcasted_iota(jnp.int32, (R, C), 1) * 128
                + jax.lax.broadcasted_iota(jnp.int32, (R, C), 0))

    for p in range(1, 11):
        m = 1 << p
        for s in [1 << q for q in range(p - 1, -1, -1)]:
            need = 'A' if s >= 128 else 'B'
            if cur != need:
                vals = [jnp.transpose(a, (1, 0)) for a in vals]
                cur = need
            R, C = vals[0].shape
            sr = s // 128 if cur == 'A' else s
            nb = R // (2 * sr)
            pos = posof(cur, R, C).reshape(nb, 2, sr, C)
            dm = (pos[:, 0] // m) % 2 == 0
            sp = [a.reshape(nb, 2, sr, C) for a in vals]
            klo, khi = sp[0][:, 0], sp[0][:, 1]
            ilo, ihi = sp[1][:, 0], sp[1][:, 1]
            before = (klo > khi) | ((klo == khi) & (ilo < ihi))
            take = dm == before
            out = []
            for a in sp:
                alo, ahi = a[:, 0], a[:, 1]
                nl = jnp.where(take, alo, ahi)
                nh = jnp.where(take, ahi, alo)
                out.append(jnp.concatenate(
                    [nl[:, None], nh[:, None]], axis=1).reshape(R, C))
            vals = out
    if cur != 'A':
        vals = [jnp.transpose(a, (1, 0)) for a in vals]
    return vals


def _k4_body(key_ref, idx_ref, keyo_ref, idxo_ref):
    k, i = _sort_net(key_ref[0], idx_ref[0], [])[:2]
    keyo_ref[0] = k
    idxo_ref[0] = i


def _k4_call(key, idx):
    spec = pl.BlockSpec((1, 8, 128), lambda b: (b, 0, 0))
    return pl.pallas_call(
        _k4_body,
        grid=(B,),
        in_specs=[spec, spec],
        out_specs=[spec, spec],
        out_shape=[jax.ShapeDtypeStruct((B, 8, 128), jnp.float32),
                   jax.ShapeDtypeStruct((B, 8, 128), jnp.int32)],
    )(key, idx)


def _compact_call(vals_flat, meta_flat, npad, unroll_static):
    chunk = npad // 8
    nrows = chunk // 128
    out_n = NSEL + 8 * chunk
    mesh = plsc.VectorSubcoreMesh(core_axis_name="c", subcore_axis_name="s")

    @functools.partial(
        pl.kernel, mesh=mesh,
        out_type=[jax.ShapeDtypeStruct((B * out_n,), jnp.float32),
                  jax.ShapeDtypeStruct((B * out_n,), jnp.int32)],
        scratch_types=[
            pltpu.VMEM((chunk,), jnp.float32),
            pltpu.VMEM((16,), jnp.int32),
            pltpu.VMEM((16,), jnp.int32),
            pltpu.VMEM((16,), jnp.int32),
            pltpu.VMEM((nrows, 1, 128), jnp.int32),
            pltpu.VMEM((nrows, 1, 128), jnp.int32),
            pltpu.VMEM((nrows, 1, 128), jnp.float32),
            pltpu.SemaphoreType.DMA,
        ],
    )
    def kfn(vals_hbm, meta_hbm, osc_hbm, oidx_hbm,
            chv, tv, cv, bv, dbuf, ibuf, sbuf, sem):
        wid = lax.axis_index("s") * 2 + lax.axis_index("c")
        b = wid // 8
        j = wid % 8
        pltpu.sync_copy(vals_hbm.at[pl.ds(b * npad + j * chunk, chunk)], chv)
        pltpu.sync_copy(meta_hbm.at[pl.ds(b * 160, 16)], tv)
        pltpu.sync_copy(meta_hbm.at[pl.ds(b * 160 + 16, 16)], cv)
        pltpu.sync_copy(meta_hbm.at[pl.ds(b * 160 + 32 + j * 16, 16)], bv)
        t = tv[...]
        cut = cv[...]
        base = bv[...]
        lane = jax.lax.iota(jnp.int32, 16)
        dump0 = b * out_n + NSEL + j * chunk
        obase = b * out_n

        def slice_step(si, run):
            sv = chv[pl.ds(si * 16, 16)]
            bits = plsc.bitcast(sv, jnp.int32)
            gi = j * chunk + si * 16 + lane
            m = (bits > t) | ((bits == t) & (gi <= cut))
            mi = m.astype(jnp.int32)
            csum = plsc.cumsum(mi)
            cnt = plsc.all_reduce_population_count(m)
            dest = jnp.where(m, obase + base + run + csum - 1,
                             dump0 + si * 16 + lane)
            r = si // 8
            c = (si % 8) * 16
            dbuf[r, 0, pl.ds(c, 16)] = dest
            ibuf[r, 0, pl.ds(c, 16)] = gi
            sbuf[r, 0, pl.ds(c, 16)] = sv
            return run + cnt

        run = jnp.zeros((16,), jnp.int32)
        if unroll_static:
            for si in range(chunk // 16):
                run = slice_step(si, run)
        else:
            run = lax.fori_loop(0, chunk // 16, slice_step, run)

        waits = []
        for r in range(nrows):
            waits.append(pltpu.async_copy(
                sbuf.at[r], osc_hbm.at[dbuf.at[r]], sem))
            waits.append(pltpu.async_copy(
                ibuf.at[r], oidx_hbm.at[dbuf.at[r]], sem))
        for w in waits:
            w.wait()

    osc, oidx = kfn(vals_flat, meta_flat)
    return osc, oidx, out_n


def _gather_rows(table, idx):
    nw = 32
    bpw = idx.shape[0] // nw
    mesh = plsc.VectorSubcoreMesh(core_axis_name="c", subcore_axis_name="s")

    @functools.partial(
        pl.kernel, mesh=mesh,
        out_type=jax.ShapeDtypeStruct((idx.shape[0], TCOLS), jnp.float32),
        scratch_types=[
            pltpu.VMEM((bpw,), jnp.int32),
            pltpu.VMEM((bpw, TCOLS), jnp.float32),
            pltpu.SemaphoreType.DMA,
        ],
    )
    def kfn(tab_hbm, idx_hbm, out_hbm, idx_v, rows_v, sem):
        wid = lax.axis_index("s") * 2 + lax.axis_index("c")
        base = wid * bpw
        pltpu.sync_copy(idx_hbm.at[pl.ds(base, bpw)], idx_v)
        pltpu.async_copy(tab_hbm.at[idx_v], rows_v, sem).wait()
        pltpu.sync_copy(rows_v, out_hbm.at[pl.ds(base, bpw)])

    return kfn(table, idx)


def _nms_body(psc_ref, pcls_ref, bx1_ref, by1_ref, bx2_ref, by2_ref, out_ref):
    sc = psc_ref[0]
    clsv = pcls_ref[0]
    x1 = bx1_ref[0]
    y1 = by1_ref[0]
    x2 = bx2_ref[0]
    y2 = by2_ref[0]
    pos = jax.lax.broadcasted_iota(jnp.int32, (1, NSEL), 1)
    lane = jax.lax.broadcasted_iota(jnp.int32, (1, 128), 1)
    areas = jnp.clip(x2 - x1, 0.0, None) * jnp.clip(y2 - y1, 0.0, None)
    BIG = jnp.int32(NSEL + 7)

    def step(t, carry):
        validf, ax1, ay1, ax2, ay2, asc, acls, aok = carry
        valid = validf > 0.0
        j = jnp.min(jnp.where(valid, pos, BIG))
        ok = j < BIG
        oh = (pos == j).astype(jnp.float32)
        bx1 = jnp.sum(x1 * oh)
        by1 = jnp.sum(y1 * oh)
        bx2 = jnp.sum(x2 * oh)
        by2 = jnp.sum(y2 * oh)
        bsc = jnp.sum(sc * oh)
        bcl = jnp.sum(clsv * oh)
        barea = jnp.clip(bx2 - bx1, 0.0, None) * jnp.clip(by2 - by1, 0.0, None)
        ix1 = jnp.maximum(bx1, x1)
        iy1 = jnp.maximum(by1, y1)
        ix2 = jnp.minimum(bx2, x2)
        iy2 = jnp.minimum(by2, y2)
        inter = jnp.clip(ix2 - ix1, 0.0, None) * jnp.clip(iy2 - iy1, 0.0, None)
        iou = inter / (barea + areas - inter + 1e-6)
        suppress = (iou > IOU_THR) & (clsv == bcl) & ok
        valid = valid & jnp.logical_not(suppress) & (pos != j)
        validf = jnp.where(valid, 1.0, 0.0)
        okf = jnp.where(ok, 1.0, 0.0)
        loh = (lane == t).astype(jnp.float32)
        ax1 = ax1 + okf * bx1 * loh
        ay1 = ay1 + okf * by1 * loh
        ax2 = ax2 + okf * bx2 * loh
        ay2 = ay2 + okf * by2 * loh
        asc = asc + okf * bsc * loh
        acls = acls + jnp.where(ok, bcl, -1.0) * loh
        aok = aok + okf * loh
        return validf, ax1, ay1, ax2, ay2, asc, acls, aok

    z = jnp.zeros((1, 128), jnp.float32)
    valid0 = jnp.where(sc > 0.0, 1.0, 0.0)
    carry = (valid0, z, z, z, z, z, z, z)
    carry = jax.lax.fori_loop(0, MAX_PER_IMG, step, carry)
    _, ax1, ay1, ax2, ay2, asc, acls, aok = carry
    num = jnp.sum(aok) * jnp.ones((1, 128), jnp.float32)
    out_ref[0] = jnp.concatenate(
        [ax1, ay1, ax2, ay2, asc, acls, aok, num], axis=0)


def _nms_call(psc, pcls, bx1, by1, bx2, by2):
    spec = pl.BlockSpec((1, 1, NSEL), lambda b: (b, 0, 0))
    return pl.pallas_call(
        _nms_body,
        grid=(B,),
        in_specs=[spec] * 6,
        out_specs=pl.BlockSpec((1, 8, 128), lambda b: (b, 0, 0)),
        out_shape=jax.ShapeDtypeStruct((B, 8, 128), jnp.float32),
    )(*[a.reshape(B, 1, NSEL) for a in (psc, pcls, bx1, by1, bx2, by2)])


def kernel(cls_score_0, cls_score_1, cls_score_2, cls_score_3, cls_score_4,
           bbox_pred_0, bbox_pred_1, bbox_pred_2, bbox_pred_3, bbox_pred_4,
           shape_pred_0, shape_pred_1, shape_pred_2, shape_pred_3, shape_pred_4,
           loc_pred_0, loc_pred_1, loc_pred_2, loc_pred_3, loc_pred_4, x):
    img_h, img_w = x.shape[2], x.shape[3]
    cls_l = [cls_score_0, cls_score_1, cls_score_2, cls_score_3, cls_score_4]
    bbox_l = [bbox_pred_0, bbox_pred_1, bbox_pred_2, bbox_pred_3, bbox_pred_4]
    shp_l = [shape_pred_0, shape_pred_1, shape_pred_2, shape_pred_3, shape_pred_4]
    loc_l = [loc_pred_0, loc_pred_1, loc_pred_2, loc_pred_3, loc_pred_4]

    maxsc_l, tab_l = [], []
    for lvl in range(5):
        ms, tab = _k1_level(lvl, cls_l[lvl], bbox_l[lvl], shp_l[lvl],
                            loc_l[lvl], img_h, img_w)
        maxsc_l.append(ms)
        tab_l.append(tab)
    maxsc = jnp.concatenate(maxsc_l, axis=1)
    table = jnp.concatenate(tab_l, axis=1)

    csc, cidx = jax.lax.top_k(maxsc, NSEL)
    tsel = jnp.take_along_axis(table, cidx[..., None], axis=1)

    col = jnp.arange(TCOLS)[None, None, :]
    prow = jnp.arange(NSEL)[None, :, None]
    valid = ((col >= 4) & (col < 4 + NC) & (prow < NMS_PRE)
             & (csc[..., None] > 0.0))
    ssv = jax.nn.sigmoid(jax.nn.sigmoid(tsel))
    pair = jnp.where(valid & (ssv >= SCORE_THR), ssv, -1.0)
    pair = pair.reshape(B, NSEL * TCOLS)

    psc, pidx = jax.lax.top_k(pair, NSEL)
    psc = jnp.where(jnp.arange(NSEL)[None, :] < NMS_PRE, psc, -1.0)
    candpos = pidx // TCOLS
    pcls = (pidx % TCOLS - 4).astype(jnp.float32)
    pbox = jnp.take_along_axis(tsel[:, :, 0:4], candpos[..., None], axis=1)

    out = _nms_call(psc, pcls, pbox[:, :, 0], pbox[:, :, 1],
                    pbox[:, :, 2], pbox[:, :, 3])
    bx = jnp.stack([out[:, 0, :MAX_PER_IMG], out[:, 1, :MAX_PER_IMG],
                    out[:, 2, :MAX_PER_IMG], out[:, 3, :MAX_PER_IMG]], axis=-1)
    scores = out[:, 4, :MAX_PER_IMG]
    cls_id = out[:, 5, :MAX_PER_IMG].astype(jnp.int32)
    num = out[:, 7, 0].astype(jnp.int32)
    return (num, bx, scores, cls_id)


# speedup vs baseline: 2.3462x; 2.3462x over previous
import functools

import jax
import jax.numpy as jnp
import numpy as np
from jax.experimental import pallas as pl

STRIDES = [8, 16, 32, 64, 128]
SIZES = [(128, 128), (64, 64), (32, 32), (16, 16), (8, 8)]
B = 4
NC = 80
TCOLS = 96
NANCH = sum(h * w for h, w in SIZES)
NPAD = 22528
NSEL = 1024
NMS_PRE = 1000
SCORE_THR = 0.05
IOU_THR = 0.5
MAX_PER_IMG = 100
MAXRATIO_G = 13.815511
MAXRATIO_P = float(abs(np.log(16.0 / 1000.0)))
LOC_THR = 0.01


def _k1_body(base, img_h, img_w, cls_ref, bbox_ref, shape_ref, loc_ref,
             px_ref, py_ref, maxsc_ref, tab_ref):
    cls = cls_ref[0]
    maxlogit = jnp.max(cls, axis=0)[None, :]
    loc_s = jax.nn.sigmoid(loc_ref[0])
    mask = loc_s >= LOC_THR
    ss = jax.nn.sigmoid(jax.nn.sigmoid(maxlogit))
    maxsc_ref[0] = jnp.where(mask, ss, 0.0)

    px = px_ref[...]
    py = py_ref[...]
    dw = jnp.clip(shape_ref[0, 0:1, :] * 0.14, -MAXRATIO_G, MAXRATIO_G)
    dh = jnp.clip(shape_ref[0, 1:2, :] * 0.14, -MAXRATIO_G, MAXRATIO_G)
    gw = base * jnp.exp(dw)
    gh = base * jnp.exp(dh)
    d2x = bbox_ref[0, 0:1, :]
    d2y = bbox_ref[0, 1:2, :]
    d2w = jnp.clip(bbox_ref[0, 2:3, :], -MAXRATIO_P, MAXRATIO_P)
    d2h = jnp.clip(bbox_ref[0, 3:4, :], -MAXRATIO_P, MAXRATIO_P)
    g2w = gw * jnp.exp(d2w)
    g2h = gh * jnp.exp(d2h)
    g2x = px + gw * d2x
    g2y = py + gh * d2y
    x1 = jnp.clip(g2x - 0.5 * g2w, 0.0, img_w)
    y1 = jnp.clip(g2y - 0.5 * g2h, 0.0, img_h)
    x2 = jnp.clip(g2x + 0.5 * g2w, 0.0, img_w)
    y2 = jnp.clip(g2y + 0.5 * g2h, 0.0, img_h)
    boxes = jnp.concatenate([x1, y1, x2, y2], axis=0)
    ch = cls.shape[1]
    row = jnp.concatenate(
        [jnp.transpose(boxes, (1, 0)),
         jnp.transpose(cls, (1, 0)),
         jnp.zeros((ch, TCOLS - 4 - NC), jnp.float32)], axis=1)
    tab_ref[0] = row


def _k1_level(lvl, cls, bbox, shp, loc, img_h, img_w):
    H, W = SIZES[lvl]
    hw = H * W
    ch = min(hw, 512)
    grid = (B, hw // ch)
    stride = STRIDES[lvl]
    xs = (np.arange(hw) % W).astype(np.float32) * stride
    ys = (np.arange(hw) // W).astype(np.float32) * stride
    px = jnp.asarray(xs)[None, :]
    py = jnp.asarray(ys)[None, :]
    base = float(stride * 4.0)
    out = pl.pallas_call(
        functools.partial(_k1_body, base, float(img_h), float(img_w)),
        grid=grid,
        in_specs=[
            pl.BlockSpec((1, NC, ch), lambda b, i: (b, 0, i)),
            pl.BlockSpec((1, 4, ch), lambda b, i: (b, 0, i)),
            pl.BlockSpec((1, 2, ch), lambda b, i: (b, 0, i)),
            pl.BlockSpec((1, 1, ch), lambda b, i: (b, 0, i)),
            pl.BlockSpec((1, ch), lambda b, i: (0, i)),
            pl.BlockSpec((1, ch), lambda b, i: (0, i)),
        ],
        out_specs=[
            pl.BlockSpec((1, 1, ch), lambda b, i: (b, 0, i)),
            pl.BlockSpec((1, ch, TCOLS), lambda b, i: (b, i, 0)),
        ],
        out_shape=[
            jax.ShapeDtypeStruct((B, 1, hw), jnp.float32),
            jax.ShapeDtypeStruct((B, hw, TCOLS), jnp.float32),
        ],
    )(cls.reshape(B, NC, hw), bbox.reshape(B, 4, hw),
      shp.reshape(B, 2, hw), loc.reshape(B, 1, hw), px, py)
    return out[0].reshape(B, hw), out[1]


def _sort_net(key, idx, extras):
    vals = [key, idx] + list(extras)
    cur = 'A'

    def posof(layout, R, C):
        if layout == 'A':
            return (jax.lax.broadcasted_iota(jnp.int32, (R, C), 0) * 128
                    + jax.lax.broadcasted_iota(jnp.int32, (R, C), 1))
        return (jax.lax.broadcasted_iota(jnp.int32, (R, C), 1) * 128
                + jax.lax.broadcasted_iota(jnp.int32, (R, C), 0))

    for p in range(1, 11):
        m = 1 << p
        for s in [1 << q for q in range(p - 1, -1, -1)]:
            need = 'A' if s >= 128 else 'B'
            if cur != need:
                vals = [jnp.transpose(a, (1, 0)) for a in vals]
                cur = need
            R, C = vals[0].shape
            sr = s // 128 if cur == 'A' else s
            nb = R // (2 * sr)
            pos = posof(cur, R, C).reshape(nb, 2, sr, C)
            dm = (pos[:, 0] // m) % 2 == 0
            sp = [a.reshape(nb, 2, sr, C) for a in vals]
            klo, khi = sp[0][:, 0], sp[0][:, 1]
            ilo, ihi = sp[1][:, 0], sp[1][:, 1]
            before = (klo > khi) | ((klo == khi) & (ilo < ihi))
            take = dm == before
            out = []
            for a in sp:
                alo, ahi = a[:, 0], a[:, 1]
                nl = jnp.where(take, alo, ahi)
                nh = jnp.where(take, ahi, alo)
                out.append(jnp.concatenate(
                    [nl[:, None], nh[:, None]], axis=1).reshape(R, C))
            vals = out
    if cur != 'A':
        vals = [jnp.transpose(a, (1, 0)) for a in vals]
    return vals


def _k4_body(key_ref, idx_ref, keyo_ref, idxo_ref):
    k, i = _sort_net(key_ref[0], idx_ref[0], [])[:2]
    keyo_ref[0] = k
    idxo_ref[0] = i


def _k4_call(key, idx):
    spec = pl.BlockSpec((1, 8, 128), lambda b: (b, 0, 0))
    return pl.pallas_call(
        _k4_body,
        grid=(B,),
        in_specs=[spec, spec],
        out_specs=[spec, spec],
        out_shape=[jax.ShapeDtypeStruct((B, 8, 128), jnp.float32),
                   jax.ShapeDtypeStruct((B, 8, 128), jnp.int32)],
    )(key, idx)


def _gather_rows(table, idx):
    return jnp.take(table, idx, axis=0)


def _nms_body(psc_ref, pcls_ref, bx1_ref, by1_ref, bx2_ref, by2_ref, out_ref):
    sc = psc_ref[0]
    clsv = pcls_ref[0]
    x1 = bx1_ref[0]
    y1 = by1_ref[0]
    x2 = bx2_ref[0]
    y2 = by2_ref[0]
    pos = jax.lax.broadcasted_iota(jnp.int32, (1, NSEL), 1)
    lane = jax.lax.broadcasted_iota(jnp.int32, (1, 128), 1)
    areas = jnp.clip(x2 - x1, 0.0, None) * jnp.clip(y2 - y1, 0.0, None)
    BIG = jnp.int32(NSEL + 7)

    def step(t, carry):
        validf, ax1, ay1, ax2, ay2, asc, acls, aok = carry
        valid = validf > 0.0
        j = jnp.min(jnp.where(valid, pos, BIG))
        ok = j < BIG
        oh = (pos == j).astype(jnp.float32)
        bx1 = jnp.sum(x1 * oh)
        by1 = jnp.sum(y1 * oh)
        bx2 = jnp.sum(x2 * oh)
        by2 = jnp.sum(y2 * oh)
        bsc = jnp.sum(sc * oh)
        bcl = jnp.sum(clsv * oh)
        barea = jnp.clip(bx2 - bx1, 0.0, None) * jnp.clip(by2 - by1, 0.0, None)
        ix1 = jnp.maximum(bx1, x1)
        iy1 = jnp.maximum(by1, y1)
        ix2 = jnp.minimum(bx2, x2)
        iy2 = jnp.minimum(by2, y2)
        inter = jnp.clip(ix2 - ix1, 0.0, None) * jnp.clip(iy2 - iy1, 0.0, None)
        iou = inter / (barea + areas - inter + 1e-6)
        suppress = (iou > IOU_THR) & (clsv == bcl) & ok
        valid = valid & jnp.logical_not(suppress) & (pos != j)
        validf = jnp.where(valid, 1.0, 0.0)
        okf = jnp.where(ok, 1.0, 0.0)
        loh = (lane == t).astype(jnp.float32)
        ax1 = ax1 + okf * bx1 * loh
        ay1 = ay1 + okf * by1 * loh
        ax2 = ax2 + okf * bx2 * loh
        ay2 = ay2 + okf * by2 * loh
        asc = asc + okf * bsc * loh
        acls = acls + jnp.where(ok, bcl, -1.0) * loh
        aok = aok + okf * loh
        return validf, ax1, ay1, ax2, ay2, asc, acls, aok

    z = jnp.zeros((1, 128), jnp.float32)
    valid0 = jnp.where(sc > 0.0, 1.0, 0.0)
    carry = (valid0, z, z, z, z, z, z, z)
    carry = jax.lax.fori_loop(0, MAX_PER_IMG, step, carry)
    _, ax1, ay1, ax2, ay2, asc, acls, aok = carry
    num = jnp.sum(aok) * jnp.ones((1, 128), jnp.float32)
    out_ref[0] = jnp.concatenate(
        [ax1, ay1, ax2, ay2, asc, acls, aok, num], axis=0)


def _nms_call(psc, pcls, bx1, by1, bx2, by2):
    spec = pl.BlockSpec((1, 1, NSEL), lambda b: (b, 0, 0))
    return pl.pallas_call(
        _nms_body,
        grid=(B,),
        in_specs=[spec] * 6,
        out_specs=pl.BlockSpec((1, 8, 128), lambda b: (b, 0, 0)),
        out_shape=jax.ShapeDtypeStruct((B, 8, 128), jnp.float32),
    )(*[a.reshape(B, 1, NSEL) for a in (psc, pcls, bx1, by1, bx2, by2)])


def _k5_body(k, vals_ref, pk_ref, key_ref, base_ref):
    v = vals_ref[0]
    R = v.shape[0]
    gidx = (jax.lax.broadcasted_iota(jnp.int32, (R, 128), 0) * 128
            + jax.lax.broadcasted_iota(jnp.int32, (R, 128), 1))
    bits = jax.lax.bitcast_convert_type(v, jnp.int32)

    def vstep(_, c):
        lo, hi = c
        mid = lo + (hi - lo + 1) // 2
        cnt = jnp.sum((bits >= mid).astype(jnp.int32))
        p = cnt >= k
        return jnp.where(p, mid, lo), jnp.where(p, hi, mid - 1)

    lo0 = jnp.int32(-1082130432)
    hi0 = jnp.int32(0x3F400000)
    t, _ = jax.lax.fori_loop(0, 31, vstep, (lo0, hi0))
    n_gt = jnp.sum((bits > t).astype(jnp.int32))
    need = k - n_gt
    eq = bits == t

    def istep(_, c):
        lo, hi = c
        mid = (lo + hi) // 2
        cnt = jnp.sum((eq & (gidx <= mid)).astype(jnp.int32))
        p = cnt >= need
        return jnp.where(p, lo, mid), jnp.where(p, mid, hi)

    _, cut = jax.lax.fori_loop(
        0, 18, istep, (jnp.int32(-1), jnp.int32(R * 128 - 1)))
    sel = (bits > t) | (eq & (gidx <= cut))

    lane = jax.lax.broadcasted_iota(jnp.int32, (R, 128), 1)
    negkey = jnp.where(sel, -lane, -(lane + 128))
    cnt = jnp.sum(sel.astype(jnp.float32), axis=1, keepdims=True)
    lt = (jax.lax.broadcasted_iota(jnp.int32, (R, R), 1)
          < jax.lax.broadcasted_iota(jnp.int32, (R, R), 0)).astype(jnp.float32)
    bases = jax.lax.dot_general(lt, cnt, (((1,), (0,)), ((), ())),
                                preferred_element_type=jnp.float32)
    base_ref[0] = jnp.broadcast_to(bases.astype(jnp.int32), (R, 128))

    kT = jnp.transpose(negkey, (1, 0))
    vT = jnp.transpose(v, (1, 0))
    pos = jax.lax.broadcasted_iota(jnp.int32, (128, R), 0)
    for p in range(1, 8):
        m = 1 << p
        for s in [1 << q for q in range(p - 1, -1, -1)]:
            nb = 128 // (2 * s)
            pr = pos.reshape(nb, 2, s, R)
            dm = (pr[:, 0] // m) % 2 == 0
            ks = kT.reshape(nb, 2, s, R)
            vs = vT.reshape(nb, 2, s, R)
            klo, khi = ks[:, 0], ks[:, 1]
            vlo, vhi = vs[:, 0], vs[:, 1]
            before = klo > khi
            take = dm == before
            knl = jnp.where(take, klo, khi)
            knh = jnp.where(take, khi, klo)
            vnl = jnp.where(take, vlo, vhi)
            vnh = jnp.where(take, vhi, vlo)
            kT = jnp.concatenate([knl[:, None], knh[:, None]],
                                 axis=1).reshape(128, R)
            vT = jnp.concatenate([vnl[:, None], vnh[:, None]],
                                 axis=1).reshape(128, R)
    pk_ref[0] = jnp.transpose(vT, (1, 0))
    key_ref[0] = -jnp.transpose(kT, (1, 0))


def _k5_call(vals):
    R = vals.shape[1]
    spec = pl.BlockSpec((1, R, 128), lambda b: (b, 0, 0))
    return pl.pallas_call(
        functools.partial(_k5_body, NSEL),
        grid=(B,),
        in_specs=[spec],
        out_specs=[spec, spec, spec],
        out_shape=[jax.ShapeDtypeStruct((B, R, 128), jnp.float32),
                   jax.ShapeDtypeStruct((B, R, 128), jnp.int32),
                   jax.ShapeDtypeStruct((B, R, 128), jnp.int32)],
    )(vals)


def _topk_select(vals, npad):
    v = vals.reshape(B, npad // 128, 128)
    pk, key, baseb = _k5_call(v)
    bases = baseb[:, :, 0]
    o = jnp.arange(NSEL, dtype=jnp.int32)
    row = jnp.sum((bases[:, :, None] <= o[None, None, :]).astype(jnp.int32),
                  axis=1) - 1
    lanepos = o[None, :] - jnp.take_along_axis(bases, row, axis=1)
    flat = row * 128 + lanepos
    val = jnp.take_along_axis(pk.reshape(B, -1), flat, axis=1)
    keyv = jnp.take_along_axis(key.reshape(B, -1), flat, axis=1)
    gidx = row * 128 + (keyv & 127)
    ssc, sidx = _k4_call(val.reshape(B, 8, 128), gidx.reshape(B, 8, 128))
    return ssc.reshape(B, NSEL), sidx.reshape(B, NSEL)


def kernel(cls_score_0, cls_score_1, cls_score_2, cls_score_3, cls_score_4,
           bbox_pred_0, bbox_pred_1, bbox_pred_2, bbox_pred_3, bbox_pred_4,
           shape_pred_0, shape_pred_1, shape_pred_2, shape_pred_3, shape_pred_4,
           loc_pred_0, loc_pred_1, loc_pred_2, loc_pred_3, loc_pred_4, x):
    img_h, img_w = x.shape[2], x.shape[3]
    cls_l = [cls_score_0, cls_score_1, cls_score_2, cls_score_3, cls_score_4]
    bbox_l = [bbox_pred_0, bbox_pred_1, bbox_pred_2, bbox_pred_3, bbox_pred_4]
    shp_l = [shape_pred_0, shape_pred_1, shape_pred_2, shape_pred_3, shape_pred_4]
    loc_l = [loc_pred_0, loc_pred_1, loc_pred_2, loc_pred_3, loc_pred_4]

    maxsc_l, tab_l = [], []
    for lvl in range(5):
        ms, tab = _k1_level(lvl, cls_l[lvl], bbox_l[lvl], shp_l[lvl],
                            loc_l[lvl], img_h, img_w)
        maxsc_l.append(ms)
        tab_l.append(tab)
    maxsc = jnp.concatenate(maxsc_l, axis=1)
    table = jnp.concatenate(tab_l, axis=1)

    pad1 = jnp.full((B, NPAD - NANCH), -1.0, jnp.float32)
    v1 = jnp.concatenate([maxsc, pad1], axis=1)
    csc, cidx = _topk_select(v1, NPAD)
    gidx = (cidx + jnp.arange(B, dtype=jnp.int32)[:, None] * NANCH)
    tsel = _gather_rows(table.reshape(B * NANCH, TCOLS),
                        gidx.reshape(-1)).reshape(B, NSEL, TCOLS)

    col = jnp.arange(TCOLS)[None, None, :]
    prow = jnp.arange(NSEL)[None, :, None]
    valid = ((col >= 4) & (col < 4 + NC) & (prow < NMS_PRE)
             & (csc[..., None] > 0.0))
    ssv = jax.nn.sigmoid(jax.nn.sigmoid(tsel))
    pair = jnp.where(valid & (ssv >= SCORE_THR), ssv, -1.0)
    pair = pair.reshape(B, NSEL * TCOLS)

    psc, pidx = _topk_select(pair, NSEL * TCOLS)
    psc = jnp.where(jnp.arange(NSEL)[None, :] < NMS_PRE, psc, -1.0)
    candpos = pidx // TCOLS
    pcls = (pidx % TCOLS - 4).astype(jnp.float32)
    pbox = jnp.take_along_axis(tsel[:, :, 0:4], candpos[..., None], axis=1)

    out = _nms_call(psc, pcls, pbox[:, :, 0], pbox[:, :, 1],
                    pbox[:, :, 2], pbox[:, :, 3])
    bx = jnp.stack([out[:, 0, :MAX_PER_IMG], out[:, 1, :MAX_PER_IMG],
                    out[:, 2, :MAX_PER_IMG], out[:, 3, :MAX_PER_IMG]], axis=-1)
    scores = out[:, 4, :MAX_PER_IMG]
    cls_id = out[:, 5, :MAX_PER_IMG].astype(jnp.int32)
    num = out[:, 7, 0].astype(jnp.int32)
    return (num, bx, scores, cls_id)
